# baseline (speedup 1.0000x reference)
import functools

import jax
import jax.numpy as jnp
from jax import lax
from jax.experimental import pallas as pl
from jax.experimental.pallas import tpu as pltpu
from jax.experimental.pallas import tpu_sc as plsc

NUM_FIELDS = 26
VOCAB = 100000
EMB = 16
BATCH = 16384
CONT = 13
OUT_W = CONT + NUM_FIELDS * EMB

_info = plsc.get_sparse_core_info()
NC, NS = _info.num_cores, _info.num_subcores
NW = NC * NS
BPW = BATCH // NW
CHUNK = 64
NCHUNK = BPW // CHUNK
IDXROWS = CHUNK * NUM_FIELDS // 128

_mesh = plsc.VectorSubcoreMesh(core_axis_name="c", subcore_axis_name="s")


@functools.partial(
    pl.kernel,
    out_type=jax.ShapeDtypeStruct((BATCH, OUT_W), jnp.float32),
    mesh=_mesh,
    compiler_params=pltpu.CompilerParams(use_tc_tiling_on_sc=False),
    scratch_types=[
        pltpu.VMEM((CHUNK * NUM_FIELDS,), jnp.int32),
        pltpu.VMEM((CHUNK * NUM_FIELDS, EMB), jnp.float32),
        pltpu.VMEM((CHUNK * CONT + 16,), jnp.float32),
        pltpu.VMEM((CHUNK, OUT_W), jnp.float32),
        pltpu.SemaphoreType.DMA,
    ],
)
def _sc_embed(gidx_hbm, xcf_hbm, tflat_hbm, out_hbm, idx_v, rows_v, cont_v, outbuf, sem):
    wid = lax.axis_index("s") * NC + lax.axis_index("c")

    def chunk_body(c, carry):
        b0 = wid * BPW + c * CHUNK
        pltpu.sync_copy(
            gidx_hbm.at[pl.ds(b0 * NUM_FIELDS, CHUNK * NUM_FIELDS)], idx_v
        )
        pltpu.sync_copy(
            xcf_hbm.at[pl.ds(b0 * CONT, CHUNK * CONT)],
            cont_v.at[pl.ds(0, CHUNK * CONT)],
        )
        copies = [
            pltpu.async_copy(
                tflat_hbm.at[idx_v.at[pl.ds(j * 128, 128)]],
                rows_v.at[pl.ds(j * 128, 128)],
                sem,
            )
            for j in range(IDXROWS)
        ]
        for cp in copies:
            cp.wait()

        def row_body(i, carry2):
            outbuf[i, pl.ds(0, 16)] = cont_v[pl.ds(i * CONT, 16)]
            for f in range(NUM_FIELDS):
                outbuf[i, pl.ds(CONT + f * EMB, EMB)] = rows_v[i * NUM_FIELDS + f, :]
            return carry2

        lax.fori_loop(0, CHUNK, row_body, 0)
        pltpu.sync_copy(outbuf, out_hbm.at[pl.ds(b0, CHUNK)])
        return carry

    lax.fori_loop(0, NCHUNK, chunk_body, 0)


def kernel(x_cont, x_cat, tables):
    offs = (jnp.arange(NUM_FIELDS, dtype=jnp.int32) * VOCAB)[None, :]
    gidx = (x_cat + offs).reshape(-1)
    xcf = x_cont.reshape(-1)
    tflat = tables.reshape(NUM_FIELDS * VOCAB, EMB)
    return _sc_embed(gidx, xcf, tflat)


# speedup vs baseline: 1.0084x; 1.0084x over previous
import functools

import jax
import jax.numpy as jnp
from jax import lax
from jax.experimental import pallas as pl
from jax.experimental.pallas import tpu as pltpu
from jax.experimental.pallas import tpu_sc as plsc

NUM_FIELDS = 26
VOCAB = 100000
EMB = 16
BATCH = 16384
CONT = 13
OUT_W = CONT + NUM_FIELDS * EMB

_info = plsc.get_sparse_core_info()
NC, NS = _info.num_cores, _info.num_subcores
NW = NC * NS
BPW = BATCH // NW
CHUNK = 128
NCHUNK = BPW // CHUNK

_mesh = plsc.VectorSubcoreMesh(core_axis_name="c", subcore_axis_name="s")


@functools.partial(
    pl.kernel,
    out_type=jax.ShapeDtypeStruct((BATCH, OUT_W), jnp.float32),
    mesh=_mesh,
    compiler_params=pltpu.CompilerParams(use_tc_tiling_on_sc=False),
    scratch_types=[
        pltpu.VMEM((NUM_FIELDS, CHUNK), jnp.int32),
        pltpu.VMEM((NUM_FIELDS * CHUNK, EMB), jnp.float32),
        pltpu.VMEM((CHUNK * CONT + 16,), jnp.float32),
        pltpu.VMEM((CHUNK, OUT_W), jnp.float32),
        pltpu.SemaphoreType.DMA,
    ],
)
def _sc_embed(xcatT_hbm, xcf_hbm, tables_hbm, out_hbm, idx_v, fgath, cont_v, outbuf, sem):
    wid = lax.axis_index("s") * NC + lax.axis_index("c")

    def chunk_body(c, carry):
        b0 = wid * BPW + c * CHUNK
        pltpu.sync_copy(xcatT_hbm.at[:, pl.ds(b0, CHUNK)], idx_v)
        pltpu.sync_copy(
            xcf_hbm.at[pl.ds(b0 * CONT, CHUNK * CONT)],
            cont_v.at[pl.ds(0, CHUNK * CONT)],
        )

        copies = [
            pltpu.async_copy(
                tables_hbm.at[f].at[idx_v.at[f]],
                fgath.at[pl.ds(f * CHUNK, CHUNK)],
                sem,
            )
            for f in range(NUM_FIELDS)
        ]
        for cp in copies:
            cp.wait()

        def row_body(i, carry2):
            outbuf[i, pl.ds(0, 16)] = cont_v[pl.ds(i * CONT, 16)]
            for f in range(NUM_FIELDS):
                outbuf[i, pl.ds(CONT + f * EMB, EMB)] = fgath[f * CHUNK + i, :]
            return carry2

        lax.fori_loop(0, CHUNK, row_body, 0)
        pltpu.sync_copy(outbuf, out_hbm.at[pl.ds(b0, CHUNK)])
        return carry

    lax.fori_loop(0, NCHUNK, chunk_body, 0)


def kernel(x_cont, x_cat, tables):
    xcatT = x_cat.T
    xcf = x_cont.reshape(-1)
    return _sc_embed(xcatT, xcf, tables)
